# baseline (speedup 1.0000x reference)
import jax
import jax.numpy as jnp
from jax.experimental import pallas as pl

K = 3
BM = 1024


def _fused_body(ps_ref, posT_ref, x_ref, w1_ref, b1_ref, w2_ref, b2_ref,
                out_ref):
    n = posT_ref.shape[1]
    bm = ps_ref.shape[1]

    pn = jnp.sum(posT_ref[...] * posT_ref[...], axis=0, keepdims=True)
    e = pn
    qn = jnp.zeros((bm, 1), dtype=jnp.float32)
    for c in range(3):
        q_c = ps_ref[c, :].reshape(bm, 1)
        p_c = posT_ref[c, :].reshape(1, n)
        e = e + q_c * (-2.0 * p_c)
        qn = qn + q_c * q_c

    s = jnp.zeros((bm, n), dtype=jnp.float32)
    wsum = jnp.zeros((bm, 1), dtype=jnp.float32)
    for k in range(K):
        m_e = jnp.min(e, axis=1, keepdims=True)
        w_k = 1.0 / jnp.maximum(m_e + qn, 1e-16)
        hit = e == m_e
        s = jnp.where(hit, w_k, s)
        if k < K - 1:
            e = jnp.where(hit, jnp.inf, e)
        wsum = wsum + w_k

    interp = jnp.dot(s.astype(jnp.bfloat16), x_ref[...],
                     preferred_element_type=jnp.float32)
    interp = interp / wsum

    h1 = jnp.dot(interp.astype(jnp.bfloat16), w1_ref[...],
                 preferred_element_type=jnp.float32)
    h1 = jnp.maximum(h1 + b1_ref[...], 0.0)
    h2 = jnp.dot(h1.astype(jnp.bfloat16), w2_ref[...],
                 preferred_element_type=jnp.float32)
    out_ref[...] = h2 + b2_ref[...]


def kernel(x, pos, x_skip, pos_skip, assign_index, W1, b1, W2, b2):
    del x_skip, assign_index
    n, d_feat = x.shape
    m = pos_skip.shape[0]
    h_feat = W2.shape[1]

    posT = pos.T
    pos_skipT = pos_skip.T
    x_bf = x.astype(jnp.bfloat16)
    w1_bf = W1.astype(jnp.bfloat16)
    w2_bf = W2.astype(jnp.bfloat16)
    b1_2d = b1.reshape(1, -1)
    b2_2d = b2.reshape(1, -1)

    grid = (m // BM,)
    out = pl.pallas_call(
        _fused_body,
        grid=grid,
        in_specs=[
            pl.BlockSpec((3, BM), lambda i: (0, i)),
            pl.BlockSpec((3, n), lambda i: (0, 0)),
            pl.BlockSpec((n, d_feat), lambda i: (0, 0)),
            pl.BlockSpec((d_feat, h_feat), lambda i: (0, 0)),
            pl.BlockSpec((1, h_feat), lambda i: (0, 0)),
            pl.BlockSpec((h_feat, h_feat), lambda i: (0, 0)),
            pl.BlockSpec((1, h_feat), lambda i: (0, 0)),
        ],
        out_specs=pl.BlockSpec((BM, h_feat), lambda i: (i, 0)),
        out_shape=jax.ShapeDtypeStruct((m, h_feat), jnp.float32),
    )(pos_skipT, posT, x_bf, w1_bf, b1_2d, w2_bf, b2_2d)

    return (out, pos_skip)
